# Initial kernel scaffold; baseline (speedup 1.0000x reference)
import jax
import jax.numpy as jnp
from jax.experimental import pallas as pl


def kernel(x_seq, edge_index, W_gcn, b_gcn, W_e1, b_e1, W_e2, b_e2, W_ew, b_ew, W_ih, W_hh, b_ih, b_hh, W_p, b_p):
    raise NotImplementedError("write your pallas kernel here")


# speedup vs baseline: 11.1375x; 11.1375x over previous
import functools

import numpy as np
import jax
import jax.numpy as jnp
from jax import lax
from jax.experimental import pallas as pl
from jax.experimental.pallas import tpu as pltpu
from jax.experimental.pallas import tpu_sc as plsc

_B, _N, _F, _H = 8, 64, 32, 64
_E = _N * (_N - 1)
_L = 16

_ii, _jj = np.meshgrid(np.arange(_N), np.arange(_N), indexing="ij")
_mask = _ii != _jj
_PERM_IDX_NP = (_ii[_mask] * _N + _jj[_mask]).astype(np.int32)


def _tc_body(x_ref, wgcn_ref, bgcn_ref, we1_ref, be1_ref, we2_ref, be2_ref,
             wew_ref, bew_ref, wp_ref, ewf_ref, q_ref):
    X = x_ref[...]
    ne = jnp.dot(X, wgcn_ref[...], preferred_element_type=jnp.float32)
    ne = ne + bgcn_ref[...]
    A = jnp.dot(ne, we1_ref[0:_H, :], preferred_element_type=jnp.float32)
    A = A + be1_ref[...]
    C = jnp.dot(ne, we1_ref[_H:2 * _H, :], preferred_element_type=jnp.float32)
    wq = jnp.dot(wew_ref[...], wp_ref[...], preferred_element_type=jnp.float32)
    cq = jnp.dot(bew_ref[...], wp_ref[...], preferred_element_type=jnp.float32)
    q = jnp.dot(ne, wq, preferred_element_type=jnp.float32) + cq
    q_ref[...] = q
    we2 = we2_ref[...]
    be2 = be2_ref[0, 0]
    for b in range(_B):
        Ab = A[b * _N:(b + 1) * _N, :]
        Cb = C[b * _N:(b + 1) * _N, :]
        T = jnp.maximum(Ab[:, None, :] + Cb[None, :, :], 0.0)
        ewb = jnp.sum(T * we2[None, :, :], axis=2) + be2
        ewf_ref[b, :, :] = ewb


_tc_call = pl.pallas_call(
    _tc_body,
    out_shape=[
        jax.ShapeDtypeStruct((_B, _N, _N), jnp.float32),
        jax.ShapeDtypeStruct((_B * _N, 1), jnp.float32),
    ],
)

_NC, _NS = 2, 16


def _sc_body(ewf_hbm, q_hbm, pidx_hbm, src_hbm, tgt_hbm, bp_hbm,
             pred_hbm, ew_hbm,
             ewf_v, q_v, pidx_v, src_v, tgt_v, ewo_v, acc_v, pred_v, bp_v):
    wid = lax.axis_index("s") * _NC + lax.axis_index("c")

    @pl.when(wid < _B)
    def _():
        b = wid
        pltpu.sync_copy(ewf_hbm.at[b], ewf_v)
        pltpu.sync_copy(q_hbm.at[b], q_v)
        pltpu.sync_copy(pidx_hbm, pidx_v)
        pltpu.sync_copy(src_hbm, src_v)
        pltpu.sync_copy(tgt_hbm, tgt_v)
        pltpu.sync_copy(bp_hbm, bp_v)

        zero = jnp.zeros((_L,), jnp.float32)

        def zbody(i, carry):
            acc_v[pl.ds(i * _L, _L)] = zero
            return carry

        lax.fori_loop(0, _L * _N // _L, zbody, 0)

        lane = lax.iota(jnp.int32, _L) * _N

        def body(i, carry):
            e0 = i * _L
            pi = pidx_v[pl.ds(e0, _L)]
            ew16 = plsc.load_gather(ewf_v, [pi])
            ewo_v[pl.ds(e0, _L)] = ew16
            si = src_v[pl.ds(e0, _L)]
            qv = plsc.load_gather(q_v, [si])
            ti = tgt_v[pl.ds(e0, _L)]
            plsc.addupdate_scatter(acc_v, [lane + ti], ew16 * qv)
            return carry

        lax.fori_loop(0, _E // _L, body, 0)

        bp = bp_v[...]
        for sl in range(_N // _L):
            s = bp
            for p in range(_L):
                s = s + acc_v[pl.ds(p * _N + sl * _L, _L)]
            pred_v[pl.ds(sl * _L, _L)] = s

        pltpu.sync_copy(ewo_v, ew_hbm.at[b])
        pltpu.sync_copy(pred_v, pred_hbm.at[b])


@functools.cache
def _make_sc_call():
    mesh = plsc.VectorSubcoreMesh(
        core_axis_name="c", subcore_axis_name="s",
        num_cores=_NC, num_subcores=_NS)
    return pl.kernel(
        _sc_body,
        out_type=[
            jax.ShapeDtypeStruct((_B, _N), jnp.float32),
            jax.ShapeDtypeStruct((_B, _E), jnp.float32),
        ],
        mesh=mesh,
        compiler_params=pltpu.CompilerParams(needs_layout_passes=False),
        scratch_types=[
            pltpu.VMEM((_N * _N,), jnp.float32),
            pltpu.VMEM((_N,), jnp.float32),
            pltpu.VMEM((_E,), jnp.int32),
            pltpu.VMEM((_E,), jnp.int32),
            pltpu.VMEM((_E,), jnp.int32),
            pltpu.VMEM((_E,), jnp.float32),
            pltpu.VMEM((_L * _N,), jnp.float32),
            pltpu.VMEM((_N,), jnp.float32),
            pltpu.VMEM((_L,), jnp.float32),
        ],
    )


def kernel(x_seq, edge_index, W_gcn, b_gcn, W_e1, b_e1, W_e2, b_e2,
           W_ew, b_ew, W_ih, W_hh, b_ih, b_hh, W_p, b_p):
    x_last = x_seq[:, -1].reshape(_B * _N, _F)
    ewf, qcol = _tc_call(
        x_last, W_gcn, b_gcn.reshape(1, _H), W_e1, b_e1.reshape(1, _H),
        W_e2.reshape(1, _H), b_e2.reshape(1, 1), W_ew, b_ew.reshape(1, _H),
        W_p)
    ewf_flat = ewf.reshape(_B, _N * _N)
    q = qcol.reshape(_B, _N)
    pidx = jnp.asarray(_PERM_IDX_NP)
    src = edge_index[0]
    tgt = edge_index[1]
    bp16 = jnp.broadcast_to(b_p.reshape(1), (_L,)).astype(jnp.float32)
    pred, ew_out = _make_sc_call()(ewf_flat, q, pidx, src, tgt, bp16)
    return pred, ew_out


# speedup vs baseline: 11.8689x; 1.0657x over previous
import functools

import numpy as np
import jax
import jax.numpy as jnp
from jax import lax
from jax.experimental import pallas as pl
from jax.experimental.pallas import tpu as pltpu
from jax.experimental.pallas import tpu_sc as plsc

_B, _W, _N, _F, _H = 8, 8, 64, 32, 64
_E = _N * (_N - 1)
_L = 16
_R, _C = 72, 128
_FLAT = _R * _C

_ii, _jj = np.meshgrid(np.arange(_N), np.arange(_N), indexing="ij")
_m = _ii != _jj
_PFLAT_NP = (_ii[_m] * _C + _jj[_m]).astype(np.int32)


def _tc_body(x_ref, wgcn_ref, bgcn_ref, we1_ref, be1_ref, we2_ref, be2_ref,
             wew_ref, bew_ref, wp_ref, bp_ref, ewf_ref):
    X = x_ref[...].reshape(_B * _N, _F)
    ne = jnp.dot(X, wgcn_ref[...], preferred_element_type=jnp.float32)
    ne = ne + bgcn_ref[...]
    A = jnp.dot(ne, we1_ref[0:_H, :], preferred_element_type=jnp.float32)
    A = A + be1_ref[...]
    C = jnp.dot(ne, we1_ref[_H:2 * _H, :], preferred_element_type=jnp.float32)
    wq = jnp.dot(wew_ref[...], wp_ref[...], preferred_element_type=jnp.float32)
    cq = jnp.dot(bew_ref[...], wp_ref[...], preferred_element_type=jnp.float32)
    q = jnp.dot(ne, wq, preferred_element_type=jnp.float32) + cq
    we2 = we2_ref[...]
    be2 = be2_ref[0, 0]
    ii = lax.broadcasted_iota(jnp.int32, (_N, _N), 0)
    jj = lax.broadcasted_iota(jnp.int32, (_N, _N), 1)
    diag = ii == jj
    bp_row = jnp.broadcast_to(bp_ref[...], (1, _N))
    for b in range(_B):
        Ab = A[b * _N:(b + 1) * _N, :]
        Cb = C[b * _N:(b + 1) * _N, :]
        T = jnp.maximum(Ab[:, None, :] + Cb[None, :, :], 0.0)
        ewb = jnp.sum(T * we2[None, :, :], axis=2) + be2
        qb = q[b * _N:(b + 1) * _N, :]
        ewf_ref[b, 0:_N, 0:_N] = jnp.where(diag, qb, ewb)
        ewf_ref[b, _N:_N + 1, 0:_N] = bp_row


_tc_call = pl.pallas_call(
    _tc_body,
    grid=(1,),
    in_specs=[pl.BlockSpec((_B, 1, _N, _F), lambda i: (0, _W - 1, 0, 0))]
    + [pl.BlockSpec(s, lambda i, _r=len(s): (0,) * _r)
       for s in [(_F, _H), (1, _H), (2 * _H, _H), (1, _H), (1, _H), (1, 1),
                 (_H, _H), (1, _H), (_H, 1), (1, 1)]],
    out_specs=pl.BlockSpec((_B, _R, _C), lambda i: (0, 0, 0)),
    out_shape=jax.ShapeDtypeStruct((_B, _R, _C), jnp.float32),
)

_NC, _NS = 2, 16


def _sc_body(ewf_hbm, eidx_hbm, pflat_hbm,
             pred_hbm, ew_hbm,
             ewf_v, pflat_v, eidx_v, ewo_v, acc_v, pred_v, sem):
    wid = lax.axis_index("s") * _NC + lax.axis_index("c")

    @pl.when(wid < _B)
    def _():
        b = wid
        copies = [
            pltpu.make_async_copy(ewf_hbm.at[b], ewf_v, sem),
            pltpu.make_async_copy(pflat_hbm, pflat_v, sem),
            pltpu.make_async_copy(eidx_hbm, eidx_v, sem),
        ]
        for c in copies:
            c.start()

        zero = jnp.zeros((_L,), jnp.float32)
        for i in range(_N):
            acc_v[pl.ds(i * _L, _L)] = zero

        for c in copies:
            c.wait()

        lane = lax.iota(jnp.int32, _L) * _N

        def body(i, carry):
            e0 = i * _L
            pf = pflat_v[pl.ds(e0, _L)]
            ew16 = plsc.load_gather(ewf_v, [pf])
            ewo_v[pl.ds(e0, _L)] = ew16
            si = eidx_v[pl.ds(e0, _L)]
            qv = plsc.load_gather(ewf_v, [si * (_C + 1)])
            ti = eidx_v[pl.ds(_E + e0, _L)]
            plsc.addupdate_scatter(acc_v, [lane + ti], ew16 * qv)
            return carry

        lax.fori_loop(0, _E // _L, body, 0)

        bp = ewf_v[pl.ds(_N * _C, _L)]
        for sl in range(_N // _L):
            s = bp
            for p in range(_L):
                s = s + acc_v[pl.ds(p * _N + sl * _L, _L)]
            pred_v[pl.ds(sl * _L, _L)] = s

        out_copies = [
            pltpu.make_async_copy(ewo_v, ew_hbm.at[b], sem),
            pltpu.make_async_copy(pred_v, pred_hbm.at[b], sem),
        ]
        for c in out_copies:
            c.start()
        for c in out_copies:
            c.wait()


@functools.cache
def _make_sc_call():
    mesh = plsc.VectorSubcoreMesh(
        core_axis_name="c", subcore_axis_name="s",
        num_cores=_NC, num_subcores=_NS)
    return pl.kernel(
        _sc_body,
        out_type=[
            jax.ShapeDtypeStruct((_B, _N), jnp.float32),
            jax.ShapeDtypeStruct((_B, _E), jnp.float32),
        ],
        mesh=mesh,
        compiler_params=pltpu.CompilerParams(needs_layout_passes=False),
        scratch_types=[
            pltpu.VMEM((_FLAT,), jnp.float32),
            pltpu.VMEM((_E,), jnp.int32),
            pltpu.VMEM((2 * _E,), jnp.int32),
            pltpu.VMEM((_E,), jnp.float32),
            pltpu.VMEM((_L * _N,), jnp.float32),
            pltpu.VMEM((_N,), jnp.float32),
            pltpu.SemaphoreType.DMA,
        ],
    )


def kernel(x_seq, edge_index, W_gcn, b_gcn, W_e1, b_e1, W_e2, b_e2,
           W_ew, b_ew, W_ih, W_hh, b_ih, b_hh, W_p, b_p):
    ewf = _tc_call(
        x_seq, W_gcn, b_gcn.reshape(1, _H), W_e1, b_e1.reshape(1, _H),
        W_e2.reshape(1, _H), b_e2.reshape(1, 1), W_ew, b_ew.reshape(1, _H),
        W_p, b_p.reshape(1, 1))
    ewf_flat = ewf.reshape(_B, _FLAT)
    eidx = jnp.concatenate([edge_index[0], edge_index[1]])
    pflat = jnp.asarray(_PFLAT_NP)
    pred, ew_out = _make_sc_call()(ewf_flat, eidx, pflat)
    return pred, ew_out
